# Initial kernel scaffold; baseline (speedup 1.0000x reference)
import jax
import jax.numpy as jnp
from jax.experimental import pallas as pl


def kernel(x, edge_index1, edge_index2, W_embed, b_embed, W1, b1, W2, b2):
    raise NotImplementedError("write your pallas kernel here")


# speedup vs baseline: 2.8459x; 2.8459x over previous
import functools

import jax
import jax.numpy as jnp
from jax import lax
from jax.experimental import pallas as pl
from jax.experimental.pallas import tpu as pltpu
from jax.experimental.pallas import tpu_sc as plsc

N = 10000
NP = 10240
D = 256
DH = 128
DA = 144
E = 160000
CHUNK = 128
CPT = 79
EPT = CHUNK * CPT
EP = EPT * 16
NR = NP // 16
FCH = 40
DUMMY = N


def _mm_aug_body(l_ref, r_ref, w_ref, b_ref, ol_ref, or_ref, *, relu):
    h = jnp.concatenate([l_ref[...], r_ref[...]], axis=1)
    h = jnp.dot(h, w_ref[...], preferred_element_type=jnp.float32) + b_ref[...]
    if relu:
        h = jnp.maximum(h, 0.0)
    br = h.shape[0]
    ones = jnp.ones((br, DA - DH), jnp.float32)
    ol_ref[...] = jnp.concatenate([h[:, :DH], ones], axis=1)
    or_ref[...] = jnp.concatenate([h[:, DH:], ones], axis=1)


def _mm_aug(hl, hr, W, b, relu):
    BR = 640
    grid = NP // BR
    return pl.pallas_call(
        functools.partial(_mm_aug_body, relu=relu),
        grid=(grid,),
        in_specs=[
            pl.BlockSpec((BR, DH), lambda i: (i, 0)),
            pl.BlockSpec((BR, DH), lambda i: (i, 0)),
            pl.BlockSpec((D, D), lambda i: (0, 0)),
            pl.BlockSpec((1, D), lambda i: (0, 0)),
        ],
        out_specs=[
            pl.BlockSpec((BR, DA), lambda i: (i, 0)),
            pl.BlockSpec((BR, DA), lambda i: (i, 0)),
        ],
        out_shape=[
            jax.ShapeDtypeStruct((NP, DA), jnp.float32),
            jax.ShapeDtypeStruct((NP, DA), jnp.float32),
        ],
    )(hl, hr, W, b.reshape(1, D))


def _mm_final_body(l_ref, r_ref, w_ref, b_ref, o_ref):
    h = jnp.concatenate([l_ref[...], r_ref[...]], axis=1)
    h = jnp.dot(h, w_ref[...], preferred_element_type=jnp.float32) + b_ref[...]
    o_ref[...] = jnp.maximum(h, 0.0)


def _mm_final(hl, hr, W, b):
    BR = 640
    grid = NP // BR
    return pl.pallas_call(
        _mm_final_body,
        grid=(grid,),
        in_specs=[
            pl.BlockSpec((BR, DH), lambda i: (i, 0)),
            pl.BlockSpec((BR, DH), lambda i: (i, 0)),
            pl.BlockSpec((D, D), lambda i: (0, 0)),
            pl.BlockSpec((1, D), lambda i: (0, 0)),
        ],
        out_specs=pl.BlockSpec((BR, D), lambda i: (i, 0)),
        out_shape=jax.ShapeDtypeStruct((NP, D), jnp.float32),
    )(hl, hr, W, b.reshape(1, D))


def _sc_agg_body(hl_hbm, hr_hbm, src_hbm, dst_hbm, z_hbm, ol_hbm, or_hbm,
                 acc, src_v, dst_v, rows_v, accv, hv, outv, sem):
    c = lax.axis_index("c")
    s = lax.axis_index("s")

    pltpu.sync_copy(z_hbm, acc.at[pl.ds(s * NR, NR)])
    plsc.subcore_barrier()

    base = s * EPT

    def edge_step(i, carry):
        off = base + i * CHUNK
        pltpu.sync_copy(src_hbm.at[pl.ds(off, CHUNK)], src_v)
        pltpu.sync_copy(dst_hbm.at[pl.ds(off, CHUNK)], dst_v)

        @pl.when(c == 0)
        def _():
            pltpu.async_copy(hl_hbm.at[src_v], rows_v, sem).wait()

        @pl.when(c == 1)
        def _():
            pltpu.async_copy(hr_hbm.at[src_v], rows_v, sem).wait()

        pltpu.sync_copy(rows_v, acc.at[dst_v], add=True)
        return carry

    lax.fori_loop(0, CPT, edge_step, 0)
    plsc.subcore_barrier()

    def fin_step(j, carry):
        r0 = s * NR + j * FCH
        pltpu.sync_copy(acc.at[pl.ds(r0, FCH)], accv)

        @pl.when(c == 0)
        def _():
            pltpu.sync_copy(hl_hbm.at[pl.ds(r0, FCH)], hv)

        @pl.when(c == 1)
        def _():
            pltpu.sync_copy(hr_hbm.at[pl.ds(r0, FCH)], hv)

        def row_step(r, carry2):
            cntb = accv[r, pl.ds(DH, 16)]
            recip = 1.0 / jnp.maximum(cntb, 1.0)
            for g in range(DH // 16):
                sl = pl.ds(g * 16, 16)
                outv[r, sl] = hv[r, sl] + accv[r, sl] * recip
            return carry2

        lax.fori_loop(0, FCH, row_step, 0)

        @pl.when(c == 0)
        def _():
            pltpu.sync_copy(outv, ol_hbm.at[pl.ds(r0, FCH)])

        @pl.when(c == 1)
        def _():
            pltpu.sync_copy(outv, or_hbm.at[pl.ds(r0, FCH)])

        return carry

    lax.fori_loop(0, NR // FCH, fin_step, 0)


def _sc_agg(hl_aug, hr_aug, src, dst, zeros_h):
    mesh = plsc.VectorSubcoreMesh(
        core_axis_name="c", subcore_axis_name="s", num_cores=2, num_subcores=16
    )
    k = pl.kernel(
        _sc_agg_body,
        out_type=[
            jax.ShapeDtypeStruct((NP, DH), jnp.float32),
            jax.ShapeDtypeStruct((NP, DH), jnp.float32),
        ],
        mesh=mesh,
        scratch_types=[
            pltpu.VMEM_SHARED((NP, DA), jnp.float32),
            pltpu.VMEM((CHUNK,), jnp.int32),
            pltpu.VMEM((CHUNK,), jnp.int32),
            pltpu.VMEM((CHUNK, DA), jnp.float32),
            pltpu.VMEM((FCH, DA), jnp.float32),
            pltpu.VMEM((FCH, DA), jnp.float32),
            pltpu.VMEM((FCH, DH), jnp.float32),
            pltpu.SemaphoreType.DMA,
        ],
        compiler_params=pltpu.CompilerParams(use_tc_tiling_on_sc=False),
    )
    return k(hl_aug, hr_aug, src, dst, zeros_h)


def _pad_edges(ei):
    pad = jnp.full((EP - E,), DUMMY, jnp.int32)
    return (jnp.concatenate([ei[0], pad]), jnp.concatenate([ei[1], pad]))


def kernel(x, edge_index1, edge_index2, W_embed, b_embed, W1, b1, W2, b2):
    xp = jnp.zeros((NP, D), jnp.float32).at[:N].set(x)
    xl, xr = xp[:, :DH], xp[:, DH:]
    src1, dst1 = _pad_edges(edge_index1)
    src2, dst2 = _pad_edges(edge_index2)
    zeros_h = jnp.zeros((NR, DA), jnp.float32)

    hl0, hr0 = _mm_aug(xl, xr, W_embed, b_embed, relu=False)
    rl1, rr1 = _sc_agg(hl0, hr0, src1, dst1, zeros_h)
    hl1, hr1 = _mm_aug(rl1, rr1, W1, b1, relu=True)
    rl2, rr2 = _sc_agg(hl1, hr1, src2, dst2, zeros_h)
    out = _mm_final(rl2, rr2, W2, b2)
    return out[:N]


# speedup vs baseline: 3.0004x; 1.0543x over previous
import functools

import jax
import jax.numpy as jnp
from jax import lax
from jax.experimental import pallas as pl
from jax.experimental.pallas import tpu as pltpu
from jax.experimental.pallas import tpu_sc as plsc

N = 10000
NP = 10240
D = 256
DH = 128
DA = 144
E = 160000
CHUNK = 128
CPT = 80
EPT = CHUNK * CPT
EP = EPT * 16
EPA = EP + CHUNK
NR = NP // 16
DUMMY = N


def _mm_aug_body(l_ref, r_ref, w_ref, b_ref, ol_ref, or_ref, *, relu):
    h = jnp.concatenate([l_ref[...], r_ref[...]], axis=1)
    h = jnp.dot(h, w_ref[...], preferred_element_type=jnp.float32) + b_ref[...]
    if relu:
        h = jnp.maximum(h, 0.0)
    br = h.shape[0]
    ones = jnp.ones((br, DA - DH), jnp.float32)
    ol_ref[...] = jnp.concatenate([h[:, :DH], ones], axis=1)
    or_ref[...] = jnp.concatenate([h[:, DH:], ones], axis=1)


def _mm_aug(hl, hr, W, b, relu):
    BR = 640
    grid = NP // BR
    return pl.pallas_call(
        functools.partial(_mm_aug_body, relu=relu),
        grid=(grid,),
        in_specs=[
            pl.BlockSpec((BR, DH), lambda i: (i, 0)),
            pl.BlockSpec((BR, DH), lambda i: (i, 0)),
            pl.BlockSpec((D, D), lambda i: (0, 0)),
            pl.BlockSpec((1, D), lambda i: (0, 0)),
        ],
        out_specs=[
            pl.BlockSpec((BR, DA), lambda i: (i, 0)),
            pl.BlockSpec((BR, DA), lambda i: (i, 0)),
        ],
        out_shape=[
            jax.ShapeDtypeStruct((NP, DA), jnp.float32),
            jax.ShapeDtypeStruct((NP, DA), jnp.float32),
        ],
    )(hl, hr, W, b.reshape(1, D))


def _rst(l_ref, r_ref, al_ref, ar_ref):
    recip = 1.0 / jnp.maximum(al_ref[:, DH:DH + 1], 1.0)
    rst_l = l_ref[:, :DH] + al_ref[:, :DH] * recip
    rst_r = r_ref[:, :DH] + ar_ref[:, :DH] * recip
    return jnp.concatenate([rst_l, rst_r], axis=1)


def _mm_layer_body(l_ref, r_ref, al_ref, ar_ref, w_ref, b_ref, ol_ref, or_ref):
    h = jnp.dot(_rst(l_ref, r_ref, al_ref, ar_ref), w_ref[...],
                preferred_element_type=jnp.float32) + b_ref[...]
    h = jnp.maximum(h, 0.0)
    ones = jnp.ones((h.shape[0], DA - DH), jnp.float32)
    ol_ref[...] = jnp.concatenate([h[:, :DH], ones], axis=1)
    or_ref[...] = jnp.concatenate([h[:, DH:], ones], axis=1)


def _mm_layer(hl, hr, al, ar, W, b):
    BR = 640
    grid = NP // BR
    spec_a = pl.BlockSpec((BR, DA), lambda i: (i, 0))
    return pl.pallas_call(
        _mm_layer_body,
        grid=(grid,),
        in_specs=[
            spec_a, spec_a, spec_a, spec_a,
            pl.BlockSpec((D, D), lambda i: (0, 0)),
            pl.BlockSpec((1, D), lambda i: (0, 0)),
        ],
        out_specs=[spec_a, spec_a],
        out_shape=[
            jax.ShapeDtypeStruct((NP, DA), jnp.float32),
            jax.ShapeDtypeStruct((NP, DA), jnp.float32),
        ],
    )(hl, hr, al, ar, W, b.reshape(1, D))


def _mm_final_body(l_ref, r_ref, al_ref, ar_ref, w_ref, b_ref, o_ref):
    h = jnp.dot(_rst(l_ref, r_ref, al_ref, ar_ref), w_ref[...],
                preferred_element_type=jnp.float32) + b_ref[...]
    o_ref[...] = jnp.maximum(h, 0.0)


def _mm_final(hl, hr, al, ar, W, b):
    BR = 640
    grid = NP // BR
    spec_a = pl.BlockSpec((BR, DA), lambda i: (i, 0))
    return pl.pallas_call(
        _mm_final_body,
        grid=(grid,),
        in_specs=[
            spec_a, spec_a, spec_a, spec_a,
            pl.BlockSpec((D, D), lambda i: (0, 0)),
            pl.BlockSpec((1, D), lambda i: (0, 0)),
        ],
        out_specs=pl.BlockSpec((BR, D), lambda i: (i, 0)),
        out_shape=jax.ShapeDtypeStruct((NP, D), jnp.float32),
    )(hl, hr, al, ar, W, b.reshape(1, D))


def _sc_agg_body(hl_hbm, hr_hbm, src_hbm, dst_hbm, z_hbm, ol_hbm, or_hbm,
                 acc, src_a, dst_a, rows_a, src_b, dst_b, rows_b,
                 sem_a, sem_b):
    c = lax.axis_index("c")
    s = lax.axis_index("s")

    pltpu.sync_copy(z_hbm, acc.at[pl.ds(s * NR, NR)])
    plsc.subcore_barrier()

    base = s * EPT

    def start_gather(src_v, rows_v, sem):
        @pl.when(c == 0)
        def _():
            pltpu.async_copy(hl_hbm.at[src_v], rows_v, sem)

        @pl.when(c == 1)
        def _():
            pltpu.async_copy(hr_hbm.at[src_v], rows_v, sem)

    def load_idx(off, src_v, dst_v):
        pltpu.sync_copy(src_hbm.at[pl.ds(off, CHUNK)], src_v)
        pltpu.sync_copy(dst_hbm.at[pl.ds(off, CHUNK)], dst_v)

    load_idx(base, src_a, dst_a)
    start_gather(src_a, rows_a, sem_a)

    def edge_step(j, carry):
        off_b = base + (2 * j + 1) * CHUNK
        off_a2 = base + (2 * j + 2) * CHUNK
        load_idx(off_b, src_b, dst_b)
        start_gather(src_b, rows_b, sem_b)
        pltpu.make_async_copy(hl_hbm.at[src_a], rows_a, sem_a).wait()
        pltpu.sync_copy(rows_a, acc.at[dst_a], add=True)
        load_idx(off_a2, src_a, dst_a)
        start_gather(src_a, rows_a, sem_a)
        pltpu.make_async_copy(hl_hbm.at[src_b], rows_b, sem_b).wait()
        pltpu.sync_copy(rows_b, acc.at[dst_b], add=True)
        return carry

    lax.fori_loop(0, CPT // 2, edge_step, 0)
    pltpu.make_async_copy(hl_hbm.at[src_a], rows_a, sem_a).wait()
    plsc.subcore_barrier()

    sl = pl.ds(s * NR, NR)

    @pl.when(c == 0)
    def _():
        pltpu.sync_copy(acc.at[sl], ol_hbm.at[sl])

    @pl.when(c == 1)
    def _():
        pltpu.sync_copy(acc.at[sl], or_hbm.at[sl])


def _sc_agg(hl_aug, hr_aug, src, dst, zeros_h):
    mesh = plsc.VectorSubcoreMesh(
        core_axis_name="c", subcore_axis_name="s", num_cores=2, num_subcores=16
    )
    k = pl.kernel(
        _sc_agg_body,
        out_type=[
            jax.ShapeDtypeStruct((NP, DA), jnp.float32),
            jax.ShapeDtypeStruct((NP, DA), jnp.float32),
        ],
        mesh=mesh,
        scratch_types=[
            pltpu.VMEM_SHARED((NP, DA), jnp.float32),
            pltpu.VMEM((CHUNK,), jnp.int32),
            pltpu.VMEM((CHUNK,), jnp.int32),
            pltpu.VMEM((CHUNK, DA), jnp.float32),
            pltpu.VMEM((CHUNK,), jnp.int32),
            pltpu.VMEM((CHUNK,), jnp.int32),
            pltpu.VMEM((CHUNK, DA), jnp.float32),
            pltpu.SemaphoreType.DMA,
            pltpu.SemaphoreType.DMA,
        ],
        compiler_params=pltpu.CompilerParams(use_tc_tiling_on_sc=False),
    )
    return k(hl_aug, hr_aug, src, dst, zeros_h)


def _pad_edges(ei):
    pad = jnp.full((EPA - E,), DUMMY, jnp.int32)
    return (jnp.concatenate([ei[0], pad]), jnp.concatenate([ei[1], pad]))


def kernel(x, edge_index1, edge_index2, W_embed, b_embed, W1, b1, W2, b2):
    xp = jnp.zeros((NP, D), jnp.float32).at[:N].set(x)
    xl, xr = xp[:, :DH], xp[:, DH:]
    src1, dst1 = _pad_edges(edge_index1)
    src2, dst2 = _pad_edges(edge_index2)
    zeros_h = jnp.zeros((NR, DA), jnp.float32)

    hl0, hr0 = _mm_aug(xl, xr, W_embed, b_embed, relu=False)
    al1, ar1 = _sc_agg(hl0, hr0, src1, dst1, zeros_h)
    hl1, hr1 = _mm_layer(hl0, hr0, al1, ar1, W1, b1)
    al2, ar2 = _sc_agg(hl1, hr1, src2, dst2, zeros_h)
    out = _mm_final(hl1, hr1, al2, ar2, W2, b2)
    return out[:N]


# speedup vs baseline: 3.0067x; 1.0021x over previous
import functools

import jax
import jax.numpy as jnp
from jax import lax
from jax.experimental import pallas as pl
from jax.experimental.pallas import tpu as pltpu
from jax.experimental.pallas import tpu_sc as plsc

N = 10000
NP = 10240
D = 256
DH = 128
DA = 144
E = 160000
CHUNK = 128
CPT = 80
EPT = CHUNK * CPT
EP = EPT * 16
EPA = EP + 2 * CHUNK
NR = NP // 16
DUMMY = N


def _mm_aug_body(l_ref, r_ref, w_ref, b_ref, ol_ref, or_ref, *, relu):
    h = jnp.concatenate([l_ref[...], r_ref[...]], axis=1)
    h = jnp.dot(h, w_ref[...], preferred_element_type=jnp.float32) + b_ref[...]
    if relu:
        h = jnp.maximum(h, 0.0)
    br = h.shape[0]
    ones = jnp.ones((br, DA - DH), jnp.float32)
    ol_ref[...] = jnp.concatenate([h[:, :DH], ones], axis=1)
    or_ref[...] = jnp.concatenate([h[:, DH:], ones], axis=1)


def _mm_aug(hl, hr, W, b, relu):
    BR = 640
    grid = NP // BR
    return pl.pallas_call(
        functools.partial(_mm_aug_body, relu=relu),
        grid=(grid,),
        in_specs=[
            pl.BlockSpec((BR, DH), lambda i: (i, 0)),
            pl.BlockSpec((BR, DH), lambda i: (i, 0)),
            pl.BlockSpec((D, D), lambda i: (0, 0)),
            pl.BlockSpec((1, D), lambda i: (0, 0)),
        ],
        out_specs=[
            pl.BlockSpec((BR, DA), lambda i: (i, 0)),
            pl.BlockSpec((BR, DA), lambda i: (i, 0)),
        ],
        out_shape=[
            jax.ShapeDtypeStruct((NP, DA), jnp.float32),
            jax.ShapeDtypeStruct((NP, DA), jnp.float32),
        ],
    )(hl, hr, W, b.reshape(1, D))


def _rst(l_ref, r_ref, al_ref, ar_ref):
    recip = 1.0 / jnp.maximum(al_ref[:, DH:DH + 1], 1.0)
    rst_l = l_ref[:, :DH] + al_ref[:, :DH] * recip
    rst_r = r_ref[:, :DH] + ar_ref[:, :DH] * recip
    return jnp.concatenate([rst_l, rst_r], axis=1)


def _mm_layer_body(l_ref, r_ref, al_ref, ar_ref, w_ref, b_ref, ol_ref, or_ref):
    h = jnp.dot(_rst(l_ref, r_ref, al_ref, ar_ref), w_ref[...],
                preferred_element_type=jnp.float32) + b_ref[...]
    h = jnp.maximum(h, 0.0)
    ones = jnp.ones((h.shape[0], DA - DH), jnp.float32)
    ol_ref[...] = jnp.concatenate([h[:, :DH], ones], axis=1)
    or_ref[...] = jnp.concatenate([h[:, DH:], ones], axis=1)


def _mm_layer(hl, hr, al, ar, W, b):
    BR = 640
    grid = NP // BR
    spec_a = pl.BlockSpec((BR, DA), lambda i: (i, 0))
    return pl.pallas_call(
        _mm_layer_body,
        grid=(grid,),
        in_specs=[
            spec_a, spec_a, spec_a, spec_a,
            pl.BlockSpec((D, D), lambda i: (0, 0)),
            pl.BlockSpec((1, D), lambda i: (0, 0)),
        ],
        out_specs=[spec_a, spec_a],
        out_shape=[
            jax.ShapeDtypeStruct((NP, DA), jnp.float32),
            jax.ShapeDtypeStruct((NP, DA), jnp.float32),
        ],
    )(hl, hr, al, ar, W, b.reshape(1, D))


def _mm_final_body(l_ref, r_ref, al_ref, ar_ref, w_ref, b_ref, o_ref):
    h = jnp.dot(_rst(l_ref, r_ref, al_ref, ar_ref), w_ref[...],
                preferred_element_type=jnp.float32) + b_ref[...]
    o_ref[...] = jnp.maximum(h, 0.0)


def _mm_final(hl, hr, al, ar, W, b):
    BR = 640
    grid = NP // BR
    spec_a = pl.BlockSpec((BR, DA), lambda i: (i, 0))
    return pl.pallas_call(
        _mm_final_body,
        grid=(grid,),
        in_specs=[
            spec_a, spec_a, spec_a, spec_a,
            pl.BlockSpec((D, D), lambda i: (0, 0)),
            pl.BlockSpec((1, D), lambda i: (0, 0)),
        ],
        out_specs=pl.BlockSpec((BR, D), lambda i: (i, 0)),
        out_shape=jax.ShapeDtypeStruct((NP, D), jnp.float32),
    )(hl, hr, al, ar, W, b.reshape(1, D))


def _sc_agg_body(hl_hbm, hr_hbm, src_hbm, dst_hbm, z_hbm, ol_hbm, or_hbm,
                 acc, src_a, dst_a, rows_a, src_b, dst_b, rows_b,
                 sem_a, sem_b, sem_ia, sem_ib):
    c = lax.axis_index("c")
    s = lax.axis_index("s")

    pltpu.sync_copy(z_hbm, acc.at[pl.ds(s * NR, NR)])
    plsc.subcore_barrier()

    base = s * EPT

    def start_gather(src_v, rows_v, sem):
        @pl.when(c == 0)
        def _():
            pltpu.async_copy(hl_hbm.at[src_v], rows_v, sem)

        @pl.when(c == 1)
        def _():
            pltpu.async_copy(hr_hbm.at[src_v], rows_v, sem)

    def start_idx(off, src_v, dst_v, sem):
        pltpu.async_copy(src_hbm.at[pl.ds(off, CHUNK)], src_v, sem)
        pltpu.async_copy(dst_hbm.at[pl.ds(off, CHUNK)], dst_v, sem)

    def wait_idx(off, src_v, dst_v, sem):
        pltpu.make_async_copy(src_hbm.at[pl.ds(off, CHUNK)], src_v, sem).wait()
        pltpu.make_async_copy(dst_hbm.at[pl.ds(off, CHUNK)], dst_v, sem).wait()

    pltpu.sync_copy(src_hbm.at[pl.ds(base, CHUNK)], src_a)
    pltpu.sync_copy(dst_hbm.at[pl.ds(base, CHUNK)], dst_a)
    start_gather(src_a, rows_a, sem_a)
    start_idx(base + CHUNK, src_b, dst_b, sem_ib)

    def edge_step(j, carry):
        off_b = base + (2 * j + 1) * CHUNK
        off_a2 = base + (2 * j + 2) * CHUNK
        off_b2 = base + (2 * j + 3) * CHUNK
        wait_idx(off_b, src_b, dst_b, sem_ib)
        start_gather(src_b, rows_b, sem_b)
        pltpu.make_async_copy(hl_hbm.at[src_a], rows_a, sem_a).wait()
        pltpu.sync_copy(rows_a, acc.at[dst_a], add=True)
        start_idx(off_a2, src_a, dst_a, sem_ia)
        pltpu.make_async_copy(hl_hbm.at[src_b], rows_b, sem_b).wait()
        wait_idx(off_a2, src_a, dst_a, sem_ia)
        start_gather(src_a, rows_a, sem_a)
        pltpu.sync_copy(rows_b, acc.at[dst_b], add=True)
        start_idx(off_b2, src_b, dst_b, sem_ib)
        return carry

    lax.fori_loop(0, CPT // 2, edge_step, 0)
    pltpu.make_async_copy(hl_hbm.at[src_a], rows_a, sem_a).wait()
    wait_idx(base, src_b, dst_b, sem_ib)
    plsc.subcore_barrier()

    sl = pl.ds(s * NR, NR)

    @pl.when(c == 0)
    def _():
        pltpu.sync_copy(acc.at[sl], ol_hbm.at[sl])

    @pl.when(c == 1)
    def _():
        pltpu.sync_copy(acc.at[sl], or_hbm.at[sl])


def _sc_agg(hl_aug, hr_aug, src, dst, zeros_h):
    mesh = plsc.VectorSubcoreMesh(
        core_axis_name="c", subcore_axis_name="s", num_cores=2, num_subcores=16
    )
    k = pl.kernel(
        _sc_agg_body,
        out_type=[
            jax.ShapeDtypeStruct((NP, DA), jnp.float32),
            jax.ShapeDtypeStruct((NP, DA), jnp.float32),
        ],
        mesh=mesh,
        scratch_types=[
            pltpu.VMEM_SHARED((NP, DA), jnp.float32),
            pltpu.VMEM((CHUNK,), jnp.int32),
            pltpu.VMEM((CHUNK,), jnp.int32),
            pltpu.VMEM((CHUNK, DA), jnp.float32),
            pltpu.VMEM((CHUNK,), jnp.int32),
            pltpu.VMEM((CHUNK,), jnp.int32),
            pltpu.VMEM((CHUNK, DA), jnp.float32),
            pltpu.SemaphoreType.DMA,
            pltpu.SemaphoreType.DMA,
            pltpu.SemaphoreType.DMA,
            pltpu.SemaphoreType.DMA,
        ],
        compiler_params=pltpu.CompilerParams(use_tc_tiling_on_sc=False),
    )
    return k(hl_aug, hr_aug, src, dst, zeros_h)


def _pad_edges(ei):
    pad = jnp.full((EPA - E,), DUMMY, jnp.int32)
    return (jnp.concatenate([ei[0], pad]), jnp.concatenate([ei[1], pad]))


def kernel(x, edge_index1, edge_index2, W_embed, b_embed, W1, b1, W2, b2):
    xp = jnp.zeros((NP, D), jnp.float32).at[:N].set(x)
    xl, xr = xp[:, :DH], xp[:, DH:]
    src1, dst1 = _pad_edges(edge_index1)
    src2, dst2 = _pad_edges(edge_index2)
    zeros_h = jnp.zeros((NR, DA), jnp.float32)

    hl0, hr0 = _mm_aug(xl, xr, W_embed, b_embed, relu=False)
    al1, ar1 = _sc_agg(hl0, hr0, src1, dst1, zeros_h)
    hl1, hr1 = _mm_layer(hl0, hr0, al1, ar1, W1, b1)
    al2, ar2 = _sc_agg(hl1, hr1, src2, dst2, zeros_h)
    out = _mm_final(hl1, hr1, al2, ar2, W2, b2)
    return out[:N]
